# baseline (speedup 1.0000x reference)
import jax
import jax.numpy as jnp
from jax import lax
from jax.experimental import pallas as pl
from jax.experimental.pallas import tpu as pltpu
from jax.experimental.pallas import tpu_sc as plsc

B = 16384
D = 128
L = 16
NC = 2
NS = 16
NW = NC * NS
BPW = B // NW
C = 32
NCHUNK = BPW // C
GROUPS = C // L
NBUF = 3
NUM_E = 100000


def _sc_body(u_id_ref, i_id_ref, user_emb, bias_cat, item_emb,
             mean_ref, out_ref,
             idx_u, idx_i, idx_ib, u_rows, i_rows, ub_rows, ib_rows, out_v,
             mean_v, sem0, sem1, sem2):
    wid = lax.axis_index("s") * NC + lax.axis_index("c")
    base_row = wid * BPW

    du = pltpu.async_copy(u_id_ref.at[pl.ds(base_row, BPW)], idx_u, sem0)
    di = pltpu.async_copy(i_id_ref.at[pl.ds(base_row, BPW)], idx_i, sem1)
    dm = pltpu.async_copy(mean_ref, mean_v.at[pl.ds(0, 1)], sem2)
    du.wait()
    di.wait()

    iota = lax.iota(jnp.int32, L)
    sems = (sem0, sem1, sem2)

    def shift_body(k, _):
        idx_ib[pl.ds(k * L, L)] = idx_i[pl.ds(k * L, L)] + NUM_E
        return 0

    lax.fori_loop(0, BPW // L, shift_body, 0)

    def start(c):
        buf = c % NBUF
        sem = sems[buf]
        iu = idx_u.at[pl.ds(c * C, C)]
        ii = idx_i.at[pl.ds(c * C, C)]
        return [
            pltpu.async_copy(user_emb.at[iu], u_rows.at[buf], sem),
            pltpu.async_copy(item_emb.at[ii], i_rows.at[buf], sem),
            pltpu.async_copy(bias_cat.at[iu], ub_rows.at[buf], sem),
            pltpu.async_copy(bias_cat.at[idx_ib.at[pl.ds(c * C, C)]],
                             ib_rows.at[buf], sem),
        ]

    pend = {0: start(0), 1: start(1)}
    dm.wait()
    m_sc = mean_v[...][0]
    for c in range(NCHUNK):
        if c + NBUF - 1 < NCHUNK:
            pend[c + NBUF - 1] = start(c + NBUF - 1)
        for d in pend.pop(c):
            d.wait()
        buf = c % NBUF
        u_b, i_b, ub_b, ib_b = (u_rows.at[buf], i_rows.at[buf],
                                ub_rows.at[buf], ib_rows.at[buf])

        @plsc.parallel_loop(0, C, step=L, carry=jnp.int32(0))
        def group_body(base, carry, u_b=u_b, i_b=i_b, ub_b=ub_b, ib_b=ib_b,
                       c=c):
            rows16 = iota + base
            UNR = 16
            init = [iota] + [jnp.zeros((L,), jnp.float32) for _ in range(8)]

            def dblock(_, carry, u_b=u_b, i_b=i_b, rows16=rows16):
                d, *accs = carry
                for j in range(UNR):
                    pu = plsc.load_gather(u_b, [rows16, d])
                    pi = plsc.load_gather(i_b, [rows16, d])
                    accs[j % 8] = accs[j % 8] + pu * pi
                    d = (d + 1) & (D - 1)
                return [d] + accs

            _, *accs = lax.fori_loop(0, D // UNR, dblock, init)
            s0 = (accs[0] + accs[1]) + (accs[2] + accs[3])
            s1 = (accs[4] + accs[5]) + (accs[6] + accs[7])
            tot = ub_b[pl.ds(base, L)] + ib_b[pl.ds(base, L)] + m_sc
            out_v[pl.ds(c * C + base, L)] = tot + (s0 + s1)
            return carry

    pltpu.sync_copy(out_v, out_ref.at[pl.ds(base_row, BPW)])


@jax.jit
def _sc_call(u_id, i_id, user_emb, bias_cat, item_emb, mean):
    mesh = plsc.VectorSubcoreMesh(core_axis_name="c", subcore_axis_name="s",
                                  num_cores=NC, num_subcores=NS)
    kern = pl.kernel(
        _sc_body,
        out_type=jax.ShapeDtypeStruct((B,), jnp.float32),
        mesh=mesh,
        compiler_params=pltpu.CompilerParams(needs_layout_passes=False),
        scratch_types=[
            pltpu.VMEM((BPW,), jnp.int32),
            pltpu.VMEM((BPW,), jnp.int32),
            pltpu.VMEM((BPW,), jnp.int32),
            pltpu.VMEM((NBUF, C, D), jnp.float32),
            pltpu.VMEM((NBUF, C, D), jnp.float32),
            pltpu.VMEM((NBUF, C), jnp.float32),
            pltpu.VMEM((NBUF, C), jnp.float32),
            pltpu.VMEM((BPW,), jnp.float32),
            pltpu.VMEM((L,), jnp.float32),
            pltpu.SemaphoreType.DMA,
            pltpu.SemaphoreType.DMA,
            pltpu.SemaphoreType.DMA,
        ],
    )
    return kern(u_id, i_id, user_emb, bias_cat, item_emb, mean)


def kernel(u_id, i_id, user_emb, user_bias, item_emb, item_bias, mean):
    bias_cat = jnp.concatenate([user_bias.reshape(-1),
                                item_bias.reshape(-1)])
    return _sc_call(u_id, i_id, user_emb, bias_cat, item_emb, mean)


# speedup vs baseline: 1.0556x; 1.0556x over previous
import jax
import jax.numpy as jnp
from jax import lax
from jax.experimental import pallas as pl
from jax.experimental.pallas import tpu as pltpu
from jax.experimental.pallas import tpu_sc as plsc

B = 16384
D = 128
L = 16
NC = 2
NS = 16
NW = NC * NS
BPW = B // NW
C = 64
NCHUNK = BPW // C
GROUPS = C // L
NBUF = 4
NUM_E = 100000


def _sc_body(u_id_ref, i_id_ref, user_emb, bias_cat, item_emb,
             mean_ref, out_ref,
             idx_u, idx_i, idx_ib, u_rows, i_rows, ub_rows, ib_rows, out_v,
             mean_v, sem0, sem1, sem2, sem3):
    wid = lax.axis_index("s") * NC + lax.axis_index("c")
    base_row = wid * BPW

    du = pltpu.async_copy(u_id_ref.at[pl.ds(base_row, BPW)], idx_u, sem0)
    di = pltpu.async_copy(i_id_ref.at[pl.ds(base_row, BPW)], idx_i, sem1)
    dm = pltpu.async_copy(mean_ref, mean_v.at[pl.ds(0, 1)], sem2)
    du.wait()
    di.wait()

    iota = lax.iota(jnp.int32, L)
    sems = (sem0, sem1, sem2, sem3)

    def shift_body(k, _):
        idx_ib[pl.ds(k * L, L)] = idx_i[pl.ds(k * L, L)] + NUM_E
        return 0

    lax.fori_loop(0, BPW // L, shift_body, 0)

    def start(c):
        buf = c % NBUF
        sem = sems[buf]
        iu = idx_u.at[pl.ds(c * C, C)]
        ii = idx_i.at[pl.ds(c * C, C)]
        return [
            pltpu.async_copy(user_emb.at[iu], u_rows.at[buf], sem),
            pltpu.async_copy(item_emb.at[ii], i_rows.at[buf], sem),
            pltpu.async_copy(bias_cat.at[iu], ub_rows.at[buf], sem),
            pltpu.async_copy(bias_cat.at[idx_ib.at[pl.ds(c * C, C)]],
                             ib_rows.at[buf], sem),
        ]

    pend = {c0: start(c0) for c0 in range(NBUF - 1)}
    dm.wait()
    m_sc = mean_v[...][0]
    for c in range(NCHUNK):
        if c + NBUF - 1 < NCHUNK:
            pend[c + NBUF - 1] = start(c + NBUF - 1)
        for d in pend.pop(c):
            d.wait()
        buf = c % NBUF
        u_b, i_b, ub_b, ib_b = (u_rows.at[buf], i_rows.at[buf],
                                ub_rows.at[buf], ib_rows.at[buf])

        @plsc.parallel_loop(0, C, step=L, carry=jnp.int32(0))
        def group_body(base, carry, u_b=u_b, i_b=i_b, ub_b=ub_b, ib_b=ib_b,
                       c=c):
            rows16 = iota + base
            UNR = 16
            init = [iota] + [jnp.zeros((L,), jnp.float32) for _ in range(8)]

            def dblock(_, carry, u_b=u_b, i_b=i_b, rows16=rows16):
                d, *accs = carry
                for j in range(UNR):
                    pu = plsc.load_gather(u_b, [rows16, d])
                    pi = plsc.load_gather(i_b, [rows16, d])
                    accs[j % 8] = accs[j % 8] + pu * pi
                    d = (d + 1) & (D - 1)
                return [d] + accs

            _, *accs = lax.fori_loop(0, D // UNR, dblock, init)
            s0 = (accs[0] + accs[1]) + (accs[2] + accs[3])
            s1 = (accs[4] + accs[5]) + (accs[6] + accs[7])
            tot = ub_b[pl.ds(base, L)] + ib_b[pl.ds(base, L)] + m_sc
            out_v[pl.ds(c * C + base, L)] = tot + (s0 + s1)
            return carry

    pltpu.sync_copy(out_v, out_ref.at[pl.ds(base_row, BPW)])


@jax.jit
def _sc_call(u_id, i_id, user_emb, bias_cat, item_emb, mean):
    mesh = plsc.VectorSubcoreMesh(core_axis_name="c", subcore_axis_name="s",
                                  num_cores=NC, num_subcores=NS)
    kern = pl.kernel(
        _sc_body,
        out_type=jax.ShapeDtypeStruct((B,), jnp.float32),
        mesh=mesh,
        compiler_params=pltpu.CompilerParams(needs_layout_passes=False),
        scratch_types=[
            pltpu.VMEM((BPW,), jnp.int32),
            pltpu.VMEM((BPW,), jnp.int32),
            pltpu.VMEM((BPW,), jnp.int32),
            pltpu.VMEM((NBUF, C, D), jnp.float32),
            pltpu.VMEM((NBUF, C, D), jnp.float32),
            pltpu.VMEM((NBUF, C), jnp.float32),
            pltpu.VMEM((NBUF, C), jnp.float32),
            pltpu.VMEM((BPW,), jnp.float32),
            pltpu.VMEM((L,), jnp.float32),
            pltpu.SemaphoreType.DMA,
            pltpu.SemaphoreType.DMA,
            pltpu.SemaphoreType.DMA,
            pltpu.SemaphoreType.DMA,
        ],
    )
    return kern(u_id, i_id, user_emb, bias_cat, item_emb, mean)


def kernel(u_id, i_id, user_emb, user_bias, item_emb, item_bias, mean):
    bias_cat = jnp.concatenate([user_bias.reshape(-1),
                                item_bias.reshape(-1)])
    return _sc_call(u_id, i_id, user_emb, bias_cat, item_emb, mean)
